# baseline (speedup 1.0000x reference)
import functools

import jax
import jax.numpy as jnp
from jax import lax
from jax.experimental import pallas as pl
from jax.experimental.pallas import tpu as pltpu
from jax.experimental.pallas import tpu_sc as plsc

_BW = 512


def _build(nb, nf, d, nc, ns):
    mesh = plsc.VectorSubcoreMesh(core_axis_name="c", subcore_axis_name="s")

    @functools.partial(
        pl.kernel,
        out_type=jax.ShapeDtypeStruct((nf, d, nb, 1), jnp.float32),
        mesh=mesh,
        scratch_types=[
            pltpu.VMEM((nf, _BW), jnp.int32),
            pltpu.VMEM((_BW, d), jnp.float32),
            pltpu.VMEM((_BW, d), jnp.float32),
            pltpu.SemaphoreType.DMA,
            pltpu.SemaphoreType.DMA,
            pltpu.SemaphoreType.DMA,
            pltpu.SemaphoreType.DMA,
        ],
        compiler_params=pltpu.CompilerParams(
            use_tc_tiling_on_sc=False, needs_layout_passes=False),
    )
    def run(idx_hbm, table_hbm, out_hbm, idx_v, a0, a1, g0, g1, w0, w1):
        wid = lax.axis_index("s") * nc + lax.axis_index("c")
        base = wid * _BW
        pltpu.sync_copy(idx_hbm.at[:, pl.ds(base, _BW)], idx_v)

        def gather(f, a, sem):
            return pltpu.async_copy(table_hbm.at[idx_v.at[f]], a, sem)

        def write(f, a, sem):
            return [
                pltpu.async_copy(
                    a.at[:, pl.ds(dd, 1)],
                    out_hbm.at[f, dd, pl.ds(base, _BW), :], sem)
                for dd in range(d)
            ]

        def drain(ws):
            for cp in ws:
                cp.wait()

        def body(i, carry):
            f0 = 2 * i
            f1 = f0 + 1
            ga0 = gather(f0, a0, g0)
            ga1 = gather(f1, a1, g1)
            ga0.wait()
            wb0 = write(f0, a0, w0)
            ga1.wait()
            wb1 = write(f1, a1, w1)
            drain(wb0)
            drain(wb1)
            return carry

        lax.fori_loop(0, nf // 2, body, 0)

    return run


def kernel(indices, weight):
    nb, nf = indices.shape
    v, d = weight.shape
    info = plsc.get_sparse_core_info()
    idx_t = indices.T.astype(jnp.int32)
    run = _build(nb, nf, d, info.num_cores, info.num_subcores)
    out_t = run(idx_t, weight)[..., 0]
    return out_t.transpose(2, 0, 1)


# speedup vs baseline: 40.3872x; 40.3872x over previous
import functools

import jax
import jax.numpy as jnp
from jax import lax
from jax.experimental import pallas as pl
from jax.experimental.pallas import tpu as pltpu
from jax.experimental.pallas import tpu_sc as plsc

_BW = 512


def _build(nb, nf, d, nc, ns):
    mesh = plsc.VectorSubcoreMesh(core_axis_name="c", subcore_axis_name="s")

    @functools.partial(
        pl.kernel,
        out_type=jax.ShapeDtypeStruct((nb, nf, d), jnp.float32),
        mesh=mesh,
        scratch_types=[
            pltpu.VMEM((nf, _BW), jnp.int32),
            pltpu.VMEM((_BW, d), jnp.float32),
            pltpu.VMEM((_BW, d), jnp.float32),
            pltpu.SemaphoreType.DMA,
            pltpu.SemaphoreType.DMA,
            pltpu.SemaphoreType.DMA,
            pltpu.SemaphoreType.DMA,
        ],
        compiler_params=pltpu.CompilerParams(
            use_tc_tiling_on_sc=False, needs_layout_passes=False),
    )
    def run(idx_hbm, table_hbm, out_hbm, idx_v, a0, a1, g0, g1, w0, w1):
        wid = lax.axis_index("s") * nc + lax.axis_index("c")
        base = wid * _BW
        pltpu.sync_copy(idx_hbm.at[:, pl.ds(base, _BW)], idx_v)

        def gather(f, a, sem):
            return pltpu.async_copy(table_hbm.at[idx_v.at[f]], a, sem)

        def write(f, a, sem):
            return pltpu.async_copy(
                a, out_hbm.at[pl.ds(base, _BW), f, :], sem)

        def body(i, carry):
            f0 = 2 * i
            f1 = f0 + 1
            ga0 = gather(f0, a0, g0)
            ga1 = gather(f1, a1, g1)
            ga0.wait()
            wb0 = write(f0, a0, w0)
            ga1.wait()
            wb1 = write(f1, a1, w1)
            wb0.wait()
            wb1.wait()
            return carry

        lax.fori_loop(0, nf // 2, body, 0)

    return run


def kernel(indices, weight):
    nb, nf = indices.shape
    v, d = weight.shape
    info = plsc.get_sparse_core_info()
    idx_t = indices.T.astype(jnp.int32)
    run = _build(nb, nf, d, info.num_cores, info.num_subcores)
    return run(idx_t, weight)


# speedup vs baseline: 42.7193x; 1.0577x over previous
import functools

import jax
import jax.numpy as jnp
from jax import lax
from jax.experimental import pallas as pl
from jax.experimental.pallas import tpu as pltpu
from jax.experimental.pallas import tpu_sc as plsc

_BW = 512


def _build(nb, nf, d, nc, ns):
    mesh = plsc.VectorSubcoreMesh(core_axis_name="c", subcore_axis_name="s")

    @functools.partial(
        pl.kernel,
        out_type=jax.ShapeDtypeStruct((nf, nb, d), jnp.float32),
        mesh=mesh,
        scratch_types=[
            pltpu.VMEM((nf, _BW), jnp.int32),
            pltpu.VMEM((_BW, d), jnp.float32),
            pltpu.VMEM((_BW, d), jnp.float32),
            pltpu.SemaphoreType.DMA,
            pltpu.SemaphoreType.DMA,
            pltpu.SemaphoreType.DMA,
            pltpu.SemaphoreType.DMA,
        ],
        compiler_params=pltpu.CompilerParams(use_tc_tiling_on_sc=False),
    )
    def run(idx_hbm, table_hbm, out_hbm, idx_v, a0, a1, g0, g1, w0, w1):
        wid = lax.axis_index("s") * nc + lax.axis_index("c")
        base = wid * _BW
        pltpu.sync_copy(idx_hbm.at[:, pl.ds(base, _BW)], idx_v)

        def gather(f, a, sem):
            return pltpu.async_copy(table_hbm.at[idx_v.at[f]], a, sem)

        def write(f, a, sem):
            return pltpu.async_copy(
                a, out_hbm.at[f, pl.ds(base, _BW), :], sem)

        def body(i, carry):
            f0 = 2 * i
            f1 = f0 + 1
            ga0 = gather(f0, a0, g0)
            ga1 = gather(f1, a1, g1)
            ga0.wait()
            wb0 = write(f0, a0, w0)
            ga1.wait()
            wb1 = write(f1, a1, w1)
            wb0.wait()
            wb1.wait()
            return carry

        lax.fori_loop(0, nf // 2, body, 0)

    return run


def kernel(indices, weight):
    nb, nf = indices.shape
    v, d = weight.shape
    info = plsc.get_sparse_core_info()
    idx_t = indices.T.astype(jnp.int32)
    run = _build(nb, nf, d, info.num_cores, info.num_subcores)
    out_f = run(idx_t, weight)
    return out_f.transpose(1, 0, 2)
